# baseline (speedup 1.0000x reference)
import functools

import jax
import jax.numpy as jnp
from jax import lax
from jax.experimental import pallas as pl
from jax.experimental.pallas import tpu as pltpu
from jax.experimental.pallas import tpu_sc as plsc

_DIM = 17
_BIN = 16.0
_NC = 2
_NS = 16


@functools.lru_cache(maxsize=None)
def _build(B, H, W, s, Wp, interpret=False):
    NW = _NC * _NS
    CW = 64
    PER_ROW = W // CW
    NGRP = CW // 16
    NIDX = CW * 8
    KIDX = NIDX // 128
    CHUNKS = B * H * PER_ROW
    CPW = CHUNKS // NW
    NPAIR = CPW // 2
    ROWS = CPW // PER_ROW
    ss = s * s
    NR = _DIM ** 4
    NRP = ((NR + _NS - 1) // _NS) * _NS
    RPT = NRP // _NS

    mesh = plsc.VectorSubcoreMesh(core_axis_name="c", subcore_axis_name="s",
                                  num_cores=_NC, num_subcores=_NS)

    @functools.partial(
        pl.kernel,
        out_type=jax.ShapeDtypeStruct((B, H, s, W, s), jnp.float32),
        mesh=mesh,
        scratch_types=[
            pltpu.VMEM_SHARED((NRP, ss), jnp.int32),
            pltpu.VMEM((ROWS + 1, Wp), jnp.float32),
            pltpu.VMEM((ROWS, W), jnp.float32),
            pltpu.VMEM((16,), jnp.float32),
            pltpu.VMEM((KIDX, 128), jnp.int32),
            pltpu.VMEM((KIDX, 128), jnp.int32),
            pltpu.VMEM((CW * 16,), jnp.float32),
            pltpu.VMEM((CW * 16,), jnp.float32),
            pltpu.VMEM((NIDX, ss), jnp.int32),
            pltpu.VMEM((NIDX, ss), jnp.int32),
            pltpu.VMEM((CW, ss), jnp.float32),
            pltpu.VMEM((CW, ss), jnp.float32),
            pltpu.SemaphoreType.DMA,
            pltpu.SemaphoreType.DMA,
            pltpu.SemaphoreType.DMA,
            pltpu.SemaphoreType.DMA,
        ],
        interpret=interpret,
        compiler_params=pltpu.CompilerParams(needs_layout_passes=False,
                                             use_tc_tiling_on_sc=False),
    )
    def sc_call(xp_hbm, w_hbm, lut_hbm, sv_hbm, out_hbm,
                slut, xband, wband, svb, idxX, idxY, wcX, wcY, rowsX, rowsY,
                obX, obY, semX, semY, semoX, semoY):
        cid = lax.axis_index("c")
        sid = lax.axis_index("s")
        wid = sid * _NC + cid
        b = wid // (NW // B)
        h0 = (wid % (NW // B)) * ROWS
        pltpu.sync_copy(lut_hbm.at[pl.ds(sid * RPT, RPT)],
                        slut.at[pl.ds(sid * RPT, RPT)])
        pltpu.sync_copy(sv_hbm, svb)
        pltpu.sync_copy(xp_hbm.at[b, pl.ds(h0, ROWS + 1)], xband)
        pltpu.sync_copy(w_hbm.at[b, pl.ds(h0, ROWS)], wband)
        plsc.subcore_barrier()
        iota = lax.iota(jnp.int32, 16)
        sv = svb[...]

        def phase_a(t, idxb, wcb):
            hl = t // PER_ROW
            w0 = (t % PER_ROW) * CW

            def ga(g, c2):
                s0 = w0 + g * 16
                x1 = xband[hl, pl.ds(s0, 16)]
                x2 = xband[hl, pl.ds(s0 + 1, 16)]
                x3 = xband[hl + 1, pl.ds(s0, 16)]
                x4 = xband[hl + 1, pl.ds(s0 + 1, 16)]
                wv = wband[hl, pl.ds(s0, 16)] * sv

                def ifr(v):
                    tq = v * (1.0 / _BIN)
                    iq = jnp.clip(tq.astype(jnp.int32), 0, _DIM - 2)
                    return iq, tq - iq.astype(jnp.float32)

                i1, f1 = ifr(x1)
                i2, f2 = ifr(x2)
                i3, f3 = ifr(x3)
                i4, f4 = ifr(x4)
                base = ((i1 * _DIM + i2) * _DIM + i3) * _DIM + i4
                g1 = 1.0 - f1
                g2 = 1.0 - f2
                g3 = 1.0 - f3
                g4 = 1.0 - f4
                p = (g1 * g2 * wv, g1 * f2 * wv, f1 * g2 * wv, f1 * f2 * wv)
                q = (g3 * g4, g3 * f4, f3 * g4, f3 * f4)
                for c in range(16):
                    d1, d2, d3, d4 = (c >> 3) & 1, (c >> 2) & 1, (c >> 1) & 1, c & 1
                    wcb[pl.ds(g * 256 + c * 16, 16)] = p[d1 * 2 + d2] * q[d3 * 2 + d4]
                    if d4 == 0:
                        cp = c >> 1
                        off = ((d1 * _DIM + d2) * _DIM + d3) * _DIM
                        idxb[g, pl.ds(cp * 16, 16)] = base + off
                return c2

            lax.fori_loop(0, NGRP, ga, None)

        def fire_gather(idxb, rows, sem):
            for kk in range(KIDX):
                pltpu.async_copy(slut.at[idxb.at[kk]],
                                 rows.at[pl.ds(kk * 128, 128)], sem)

        def wait_gather(idxb, rows, sem):
            for kk in range(KIDX):
                pltpu.make_async_copy(slut.at[idxb.at[kk]],
                                      rows.at[pl.ds(kk * 128, 128)], sem).wait()

        def out_slices(t):
            h = h0 + t // PER_ROW
            w0 = (t % PER_ROW) * CW
            return [out_hbm.at[b, h, r, pl.ds(w0, CW), :] for r in range(s)]

        def phase_b(t, wcb, rows, ob, semo, first):
            dsts = out_slices(t)

            @pl.when(jnp.logical_not(first))
            def _():
                for r in range(s):
                    pltpu.make_async_copy(ob.at[:, pl.ds(r * s, s)],
                                          dsts[r], semo).wait()

            def gb(g, c2):
                acc = [jnp.zeros((16,), jnp.float32) for _ in range(ss)]
                for cp in range(8):
                    nbase = g * 256 + cp * 32
                    wc0 = wcb[pl.ds(nbase, 16)]
                    wc1 = wcb[pl.ds(nbase + 16, 16)]
                    rb = iota + (g * 128 + cp * 16)
                    for j in range(ss):
                        rotv = lax.bitwise_and(iota + j, 15)
                        w = plsc.load_gather(rows, [rb, rotv])
                        lo = plsc.bitcast(lax.shift_left(w, 16), jnp.float32)
                        hi = plsc.bitcast(
                            lax.bitwise_and(w, jnp.int32(-65536)), jnp.float32)
                        acc[j] = acc[j] + wc0 * lo + wc1 * hi
                pix = iota + g * 16
                for j in range(ss):
                    rotv = lax.bitwise_and(iota + j, 15)
                    plsc.store_scatter(ob, [pix, rotv], acc[j])
                return c2

            lax.fori_loop(0, NGRP, gb, None)
            for r in range(s):
                pltpu.async_copy(ob.at[:, pl.ds(r * s, s)], dsts[r], semo)

        phase_a(0, idxX, wcX)
        fire_gather(idxX, rowsX, semX)

        def pair_body(i, carry):
            t = 2 * i
            phase_a(t + 1, idxY, wcY)
            fire_gather(idxY, rowsY, semY)
            wait_gather(idxX, rowsX, semX)
            phase_b(t, wcX, rowsX, obX, semoX, i == 0)

            @pl.when(i < NPAIR - 1)
            def _():
                phase_a(t + 2, idxX, wcX)
                fire_gather(idxX, rowsX, semX)

            wait_gather(idxY, rowsY, semY)
            phase_b(t + 1, wcY, rowsY, obY, semoY, i == 0)
            return carry

        lax.fori_loop(0, NPAIR, pair_body, None)
        for r in range(s):
            pltpu.make_async_copy(obX.at[:, pl.ds(r * s, s)],
                                  out_slices(CPW - 2)[r], semoX).wait()
            pltpu.make_async_copy(obY.at[:, pl.ds(r * s, s)],
                                  out_slices(CPW - 1)[r], semoY).wait()

    return sc_call


def kernel(weight, x, scale_factor, LUTs, tri_index):
    B, C, H, W = x.shape
    L, dim = LUTs.shape[0], LUTs.shape[1]
    s = LUTs.shape[-1]
    assert C == 1 and L == 1 and dim == _DIM
    assert W % 64 == 0 and (B * H * 2) % (_NC * _NS) == 0

    xp = jnp.pad(x[:, 0], ((0, 0), (0, 1), (0, 1)), mode="reflect")
    Wp = ((W + 1 + 7) // 8) * 8
    xp = jnp.pad(xp, ((0, 0), (0, 0), (0, Wp - (W + 1))))
    w3 = weight[:, 0]
    lut_flat = LUTs.reshape(dim ** 4, s * s)
    nrp = ((dim ** 4 + _NS - 1) // _NS) * _NS
    lut_flat = jnp.pad(lut_flat, ((0, nrp - dim ** 4), (0, 0)))
    lut_bf = lut_flat.astype(jnp.bfloat16)
    lut_shift = jnp.concatenate(
        [lut_bf[1:], jnp.zeros((1, s * s), jnp.bfloat16)], axis=0)
    lut_pair = jnp.stack([lut_bf, lut_shift], axis=-1)
    lut_i32 = jax.lax.bitcast_convert_type(lut_pair, jnp.int32)
    sv = jnp.full((16,), scale_factor, jnp.float32) * (1.0 / s)

    out5 = _build(B, H, W, s, Wp)(xp, w3, lut_i32, sv)
    return out5.reshape(B, 1, H * s, W * s)


# speedup vs baseline: 7.3680x; 7.3680x over previous
import functools

import jax
import jax.numpy as jnp
from jax import lax
from jax.experimental import pallas as pl
from jax.experimental.pallas import tpu as pltpu
from jax.experimental.pallas import tpu_sc as plsc

_DIM = 17
_BIN = 16.0
_NC = 2
_NS = 16


@functools.lru_cache(maxsize=None)
def _build(B, H, W, s, Wp, interpret=False):
    NW = _NC * _NS
    CW = 64
    PER_ROW = W // CW
    NGRP = CW // 16
    NIDX = CW * 8
    KIDX = NIDX // 128
    CHUNKS = B * H * PER_ROW
    CPW = CHUNKS // NW
    NPAIR = CPW // 2
    ROWS = CPW // PER_ROW
    ss = s * s
    NR = _DIM ** 4
    NRP = ((NR + _NS - 1) // _NS) * _NS
    RPT = NRP // _NS

    mesh = plsc.VectorSubcoreMesh(core_axis_name="c", subcore_axis_name="s",
                                  num_cores=_NC, num_subcores=_NS)

    @functools.partial(
        pl.kernel,
        out_type=jax.ShapeDtypeStruct((B, H, s, s * W), jnp.float32),
        mesh=mesh,
        scratch_types=[
            pltpu.VMEM_SHARED((NRP, ss), jnp.int32),
            pltpu.VMEM((ROWS + 1, Wp), jnp.float32),
            pltpu.VMEM((ROWS, W), jnp.float32),
            pltpu.VMEM((16,), jnp.float32),
            pltpu.VMEM((KIDX, 128), jnp.int32),
            pltpu.VMEM((KIDX, 128), jnp.int32),
            pltpu.VMEM((CW * 16,), jnp.float32),
            pltpu.VMEM((CW * 16,), jnp.float32),
            pltpu.VMEM((NIDX, ss), jnp.int32),
            pltpu.VMEM((NIDX, ss), jnp.int32),
            pltpu.VMEM((s, s * CW), jnp.float32),
            pltpu.VMEM((s, s * CW), jnp.float32),
            pltpu.SemaphoreType.DMA,
            pltpu.SemaphoreType.DMA,
            pltpu.SemaphoreType.DMA,
            pltpu.SemaphoreType.DMA,
        ],
        interpret=interpret,
        compiler_params=pltpu.CompilerParams(needs_layout_passes=False,
                                             use_tc_tiling_on_sc=False),
    )
    def sc_call(xp_hbm, w_hbm, lut_hbm, sv_hbm, out_hbm,
                slut, xband, wband, svb, idxX, idxY, wcX, wcY, rowsX, rowsY,
                obX, obY, semX, semY, semoX, semoY):
        cid = lax.axis_index("c")
        sid = lax.axis_index("s")
        wid = sid * _NC + cid
        b = wid // (NW // B)
        h0 = (wid % (NW // B)) * ROWS
        pltpu.sync_copy(lut_hbm.at[pl.ds(sid * RPT, RPT)],
                        slut.at[pl.ds(sid * RPT, RPT)])
        pltpu.sync_copy(sv_hbm, svb)
        pltpu.sync_copy(xp_hbm.at[b, pl.ds(h0, ROWS + 1)], xband)
        pltpu.sync_copy(w_hbm.at[b, pl.ds(h0, ROWS)], wband)
        plsc.subcore_barrier()
        iota = lax.iota(jnp.int32, 16)
        sv = svb[...]

        def phase_a(t, idxb, wcb):
            hl = t // PER_ROW
            w0 = (t % PER_ROW) * CW

            def ga(g, c2):
                s0 = w0 + g * 16
                x1 = xband[hl, pl.ds(s0, 16)]
                x2 = xband[hl, pl.ds(s0 + 1, 16)]
                x3 = xband[hl + 1, pl.ds(s0, 16)]
                x4 = xband[hl + 1, pl.ds(s0 + 1, 16)]
                wv = wband[hl, pl.ds(s0, 16)] * sv

                def ifr(v):
                    tq = v * (1.0 / _BIN)
                    iq = jnp.clip(tq.astype(jnp.int32), 0, _DIM - 2)
                    return iq, tq - iq.astype(jnp.float32)

                i1, f1 = ifr(x1)
                i2, f2 = ifr(x2)
                i3, f3 = ifr(x3)
                i4, f4 = ifr(x4)
                base = ((i1 * _DIM + i2) * _DIM + i3) * _DIM + i4
                g1 = 1.0 - f1
                g2 = 1.0 - f2
                g3 = 1.0 - f3
                g4 = 1.0 - f4
                p = (g1 * g2 * wv, g1 * f2 * wv, f1 * g2 * wv, f1 * f2 * wv)
                q = (g3 * g4, g3 * f4, f3 * g4, f3 * f4)
                for c in range(16):
                    d1, d2, d3, d4 = (c >> 3) & 1, (c >> 2) & 1, (c >> 1) & 1, c & 1
                    wcb[pl.ds(g * 256 + c * 16, 16)] = p[d1 * 2 + d2] * q[d3 * 2 + d4]
                    if d4 == 0:
                        cp = c >> 1
                        off = ((d1 * _DIM + d2) * _DIM + d3) * _DIM
                        idxb[g, pl.ds(cp * 16, 16)] = base + off
                return c2

            lax.fori_loop(0, NGRP, ga, None)

        def fire_gather(idxb, rows, sem):
            for kk in range(KIDX):
                pltpu.async_copy(slut.at[idxb.at[kk]],
                                 rows.at[pl.ds(kk * 128, 128)], sem)

        def wait_gather(idxb, rows, sem):
            for kk in range(KIDX):
                pltpu.make_async_copy(slut.at[idxb.at[kk]],
                                      rows.at[pl.ds(kk * 128, 128)], sem).wait()

        def out_slice(t):
            return out_hbm.at[b, h0 + t // PER_ROW, :,
                              pl.ds((t % PER_ROW) * s * CW, s * CW)]

        def phase_b(t, wcb, rows, ob, semo, first):
            @pl.when(jnp.logical_not(first))
            def _():
                pltpu.make_async_copy(ob, out_slice(t), semo).wait()

            def gb(g, c2):
                acc = [jnp.zeros((16,), jnp.float32) for _ in range(ss)]
                for cp in range(8):
                    nbase = g * 256 + cp * 32
                    wc0 = wcb[pl.ds(nbase, 16)]
                    wc1 = wcb[pl.ds(nbase + 16, 16)]
                    rb = iota + (g * 128 + cp * 16)
                    for j in range(ss):
                        w = plsc.load_gather(
                            rows, [rb, jnp.full((16,), j, jnp.int32)])
                        lo = plsc.bitcast(lax.shift_left(w, 16), jnp.float32)
                        hi = plsc.bitcast(
                            lax.bitwise_and(w, jnp.int32(-65536)), jnp.float32)
                        acc[j] = acc[j] + wc0 * lo + wc1 * hi
                for j in range(ss):
                    r = j // s
                    cc = j % s
                    colidx = iota * s + (g * 16 * s + cc)
                    plsc.store_scatter(
                        ob, [jnp.full((16,), r, jnp.int32), colidx], acc[j])
                return c2

            lax.fori_loop(0, NGRP, gb, None)
            pltpu.async_copy(ob, out_slice(t), semo)

        phase_a(0, idxX, wcX)
        fire_gather(idxX, rowsX, semX)

        def pair_body(i, carry):
            t = 2 * i
            phase_a(t + 1, idxY, wcY)
            fire_gather(idxY, rowsY, semY)
            wait_gather(idxX, rowsX, semX)
            phase_b(t, wcX, rowsX, obX, semoX, i == 0)

            @pl.when(i < NPAIR - 1)
            def _():
                phase_a(t + 2, idxX, wcX)
                fire_gather(idxX, rowsX, semX)

            wait_gather(idxY, rowsY, semY)
            phase_b(t + 1, wcY, rowsY, obY, semoY, i == 0)
            return carry

        lax.fori_loop(0, NPAIR, pair_body, None)
        pltpu.make_async_copy(obX, out_slice(CPW - 2), semoX).wait()
        pltpu.make_async_copy(obY, out_slice(CPW - 1), semoY).wait()

    return sc_call


def kernel(weight, x, scale_factor, LUTs, tri_index):
    B, C, H, W = x.shape
    L, dim = LUTs.shape[0], LUTs.shape[1]
    s = LUTs.shape[-1]
    assert C == 1 and L == 1 and dim == _DIM
    assert W % 64 == 0 and (B * H * 2) % (_NC * _NS) == 0

    xp = jnp.pad(x[:, 0], ((0, 0), (0, 1), (0, 1)), mode="reflect")
    Wp = ((W + 1 + 7) // 8) * 8
    xp = jnp.pad(xp, ((0, 0), (0, 0), (0, Wp - (W + 1))))
    w3 = weight[:, 0]
    lut_flat = LUTs.reshape(dim ** 4, s * s)
    nrp = ((dim ** 4 + _NS - 1) // _NS) * _NS
    lut_flat = jnp.pad(lut_flat, ((0, nrp - dim ** 4), (0, 0)))
    lut_bf = lut_flat.astype(jnp.bfloat16)
    lut_shift = jnp.concatenate(
        [lut_bf[1:], jnp.zeros((1, s * s), jnp.bfloat16)], axis=0)
    lut_pair = jnp.stack([lut_bf, lut_shift], axis=-1)
    lut_i32 = jax.lax.bitcast_convert_type(lut_pair, jnp.int32)
    sv = jnp.full((16,), scale_factor, jnp.float32) * (1.0 / s)

    out5 = _build(B, H, W, s, Wp)(xp, w3, lut_i32, sv)
    return out5.reshape(B, 1, H * s, W * s)
